# baseline (speedup 1.0000x reference)
import functools

import jax
import jax.numpy as jnp
from jax import lax
from jax.experimental import pallas as pl
from jax.experimental.pallas import tpu as pltpu
from jax.experimental.pallas import tpu_sc as plsc

N = 320000
D_IN = 128
D_HID = 256
S = 10000

PARTS = (179200, 140800)
RB = 12800
NC = 2
NS = 16
NW = NC * NS
K = 128
NBUF = 3
SEG_PER_TILE = 624
SEG_TAIL = S - NS * SEG_PER_TILE

PART_OFF = tuple(sum(PARTS[:i]) for i in range(len(PARTS)))
assert sum(PARTS) == N
for _p in PARTS:
    _rpw = _p // NW
    assert _p % NW == 0 and _rpw % 8 == 0 and _p % RB == 0
    assert (_rpw - (_rpw // K) * K) % 8 == 0


def _z_body(x_ref, w1t_ref, b1_ref, w2t_ref, z_ref):
    xb = x_ref[...].astype(jnp.bfloat16)
    h = jnp.dot(xb, w1t_ref[...], preferred_element_type=jnp.float32)
    phi = jnp.tanh(h + b1_ref[...])
    z_ref[...] = jnp.dot(phi.astype(jnp.bfloat16), w2t_ref[...],
                         preferred_element_type=jnp.float32)


def _compute_z(x, W1T, b1, W2T, part):
    npart = PARTS[part]
    nb0 = PART_OFF[part] // RB
    return pl.pallas_call(
        _z_body,
        grid=(npart // RB,),
        in_specs=[
            pl.BlockSpec((RB, D_IN), lambda i: (i + nb0, 0)),
            pl.BlockSpec((D_IN, D_HID), lambda i: (0, 0)),
            pl.BlockSpec((1, D_HID), lambda i: (0, 0)),
            pl.BlockSpec((D_HID, D_IN), lambda i: (0, 0)),
        ],
        out_specs=pl.BlockSpec((RB, D_IN), lambda i: (i, 0)),
        out_shape=jax.ShapeDtypeStruct((npart, D_IN), jnp.float32),
    )(x, W1T, b1.reshape(1, D_HID), W2T)


def _sc_scatter_body(part, z_hbm, b_hbm, zero_hbm, out_hbm,
                     zb0, zb1, zb2, ib0, ib1, ib2, tbuf, acc,
                     zs0, zs1, zs2, is0, is1, is2):
    rpw = PARTS[part] // NW
    nk = rpw // K
    ktail = rpw - nk * K

    c = lax.axis_index("c")
    s = lax.axis_index("s")
    w = c * NS + s
    row_base = w * rpw
    id_base = PART_OFF[part] + w * rpw
    zbufs = (zb0, zb1, zb2)
    ibufs = (ib0, ib1, ib2)
    zsems = (zs0, zs1, zs2)
    isems = (is0, is1, is2)

    zr0 = pl.multiple_of(s * SEG_PER_TILE, 8)
    pltpu.sync_copy(zero_hbm.at[pl.ds(zr0, SEG_PER_TILE)],
                    acc.at[pl.ds(zr0, SEG_PER_TILE)])

    @pl.when(s == NS - 1)
    def _zero_tail():
        t0 = pl.multiple_of(NS * SEG_PER_TILE, 8)
        pltpu.sync_copy(zero_hbm.at[pl.ds(t0, SEG_TAIL)],
                        acc.at[pl.ds(t0, SEG_TAIL)])

    plsc.subcore_barrier()

    def zsrc(l):
        return z_hbm.at[pl.ds(pl.multiple_of(row_base + l * K, 8), K)]

    def isrc(l):
        return b_hbm.at[pl.ds(pl.multiple_of(id_base + l * K, 8), K)]

    def wait_and_scatter(l, b):
        pltpu.make_async_copy(zsrc(l), zbufs[b], zsems[b]).wait()
        pltpu.make_async_copy(isrc(l), ibufs[b], isems[b]).wait()
        pltpu.sync_copy(zbufs[b], acc.at[ibufs[b]], add=True)

    for b in range(NBUF):
        pltpu.async_copy(zsrc(b), zbufs[b], zsems[b])
        pltpu.async_copy(isrc(b), ibufs[b], isems[b])

    def body(i, carry):
        for b in range(NBUF):
            l = i * NBUF + b
            wait_and_scatter(l, b)
            nl = l + NBUF

            @pl.when(nl < nk)
            def _next():
                pltpu.async_copy(zsrc(nl), zbufs[b], zsems[b])
                pltpu.async_copy(isrc(nl), ibufs[b], isems[b])

        return carry

    lax.fori_loop(0, nk // NBUF, body, 0)
    for l in range(nk - nk % NBUF, nk):
        wait_and_scatter(l, l % NBUF)

    if ktail:
        t0 = pl.multiple_of(row_base + nk * K, 8)
        ti0 = pl.multiple_of(id_base + nk * K, 8)
        pltpu.sync_copy(z_hbm.at[pl.ds(t0, ktail)], zb0.at[pl.ds(0, ktail)])
        pltpu.sync_copy(b_hbm.at[pl.ds(ti0, ktail)], tbuf.at[pl.ds(0, ktail)])
        pltpu.sync_copy(zb0.at[pl.ds(0, ktail)],
                        acc.at[tbuf.at[pl.ds(0, ktail)]], add=True)

    plsc.subcore_barrier()

    pltpu.sync_copy(acc.at[pl.ds(zr0, SEG_PER_TILE)],
                    out_hbm.at[c].at[pl.ds(zr0, SEG_PER_TILE)])

    @pl.when(s == NS - 1)
    def _write_tail():
        t0w = pl.multiple_of(NS * SEG_PER_TILE, 8)
        pltpu.sync_copy(acc.at[pl.ds(t0w, SEG_TAIL)],
                        out_hbm.at[c].at[pl.ds(t0w, SEG_TAIL)])


def _sc_scatter(z, bidx, zero, part):
    mesh = plsc.VectorSubcoreMesh(
        core_axis_name="c", subcore_axis_name="s", num_cores=NC, num_subcores=NS
    )
    return pl.kernel(
        functools.partial(_sc_scatter_body, part),
        out_type=jax.ShapeDtypeStruct((NC, S, D_IN), jnp.float32),
        mesh=mesh,
        scratch_types=[
            pltpu.VMEM((K, D_IN), jnp.float32),
            pltpu.VMEM((K, D_IN), jnp.float32),
            pltpu.VMEM((K, D_IN), jnp.float32),
            pltpu.VMEM((K,), jnp.int32),
            pltpu.VMEM((K,), jnp.int32),
            pltpu.VMEM((K,), jnp.int32),
            pltpu.VMEM((K,), jnp.int32),
            pltpu.VMEM_SHARED((S, D_IN), jnp.float32),
            pltpu.SemaphoreType.DMA,
            pltpu.SemaphoreType.DMA,
            pltpu.SemaphoreType.DMA,
            pltpu.SemaphoreType.DMA,
            pltpu.SemaphoreType.DMA,
            pltpu.SemaphoreType.DMA,
        ],
    )(z, bidx, zero)


def _combine_body(*refs):
    p_refs, b2_ref, o_ref = refs[:-2], refs[-2], refs[-1]
    total = b2_ref[...]
    for p in p_refs:
        total = total + p[0] + p[1]
    o_ref[...] = total


def _combine(partials, b2):
    return pl.pallas_call(
        _combine_body,
        grid=(10,),
        in_specs=[pl.BlockSpec((NC, S // 10, D_IN), lambda i: (0, i, 0))
                  for _ in partials]
        + [pl.BlockSpec((1, D_IN), lambda i: (0, 0))],
        out_specs=pl.BlockSpec((S // 10, D_IN), lambda i: (i, 0)),
        out_shape=jax.ShapeDtypeStruct((S, D_IN), jnp.float32),
    )(*partials, b2.reshape(1, D_IN))


def kernel(x, batch, W1, b1, W2, b2):
    bidx = batch.astype(jnp.int32)
    w1t = W1.T.astype(jnp.bfloat16)
    w2t = W2.T.astype(jnp.bfloat16)
    zero = jnp.zeros((S, D_IN), jnp.float32)
    partials = []
    for part in range(len(PARTS)):
        z = _compute_z(x, w1t, b1, w2t, part)
        partials.append(_sc_scatter(z, bidx, zero, part))
    return _combine(partials, b2)


# speedup vs baseline: 1.0242x; 1.0242x over previous
import functools

import jax
import jax.numpy as jnp
from jax import lax
from jax.experimental import pallas as pl
from jax.experimental.pallas import tpu as pltpu
from jax.experimental.pallas import tpu_sc as plsc

N = 320000
D_IN = 128
D_HID = 256
S = 10000

PARTS = (160000, 160000)
RB = 16000
NC = 2
NS = 16
NW = NC * NS
K = 128
NBUF = 3
SEG_PER_TILE = 624
SEG_TAIL = S - NS * SEG_PER_TILE

PART_OFF = tuple(sum(PARTS[:i]) for i in range(len(PARTS)))
assert sum(PARTS) == N
for _p in PARTS:
    _rpw = _p // NW
    assert _p % NW == 0 and _rpw % 8 == 0 and _p % RB == 0
    assert (_rpw - (_rpw // K) * K) % 8 == 0


def _z_body(x_ref, w1t_ref, b1_ref, w2t_ref, z_ref):
    xb = x_ref[...].astype(jnp.bfloat16)
    h = jnp.dot(xb, w1t_ref[...], preferred_element_type=jnp.float32)
    phi = jnp.tanh(h + b1_ref[...])
    z_ref[...] = jnp.dot(phi.astype(jnp.bfloat16), w2t_ref[...],
                         preferred_element_type=jnp.float32)


def _compute_z(x, W1T, b1, W2T, part):
    npart = PARTS[part]
    nb0 = PART_OFF[part] // RB
    return pl.pallas_call(
        _z_body,
        grid=(npart // RB,),
        in_specs=[
            pl.BlockSpec((RB, D_IN), lambda i: (i + nb0, 0)),
            pl.BlockSpec((D_IN, D_HID), lambda i: (0, 0)),
            pl.BlockSpec((1, D_HID), lambda i: (0, 0)),
            pl.BlockSpec((D_HID, D_IN), lambda i: (0, 0)),
        ],
        out_specs=pl.BlockSpec((RB, D_IN), lambda i: (i, 0)),
        out_shape=jax.ShapeDtypeStruct((npart, D_IN), jnp.float32),
    )(x, W1T, b1.reshape(1, D_HID), W2T)


def _sc_scatter_body(part, z_hbm, b_hbm, zero_hbm, out_hbm,
                     zb0, zb1, zb2, ib0, ib1, ib2, tbuf, acc,
                     zs0, zs1, zs2, is0, is1, is2):
    rpw = PARTS[part] // NW
    nk = rpw // K
    ktail = rpw - nk * K

    c = lax.axis_index("c")
    s = lax.axis_index("s")
    w = c * NS + s
    row_base = w * rpw
    id_base = PART_OFF[part] + w * rpw
    zbufs = (zb0, zb1, zb2)
    ibufs = (ib0, ib1, ib2)
    zsems = (zs0, zs1, zs2)
    isems = (is0, is1, is2)

    zr0 = pl.multiple_of(s * SEG_PER_TILE, 8)
    pltpu.sync_copy(zero_hbm.at[pl.ds(zr0, SEG_PER_TILE)],
                    acc.at[pl.ds(zr0, SEG_PER_TILE)])

    @pl.when(s == NS - 1)
    def _zero_tail():
        t0 = pl.multiple_of(NS * SEG_PER_TILE, 8)
        pltpu.sync_copy(zero_hbm.at[pl.ds(t0, SEG_TAIL)],
                        acc.at[pl.ds(t0, SEG_TAIL)])

    plsc.subcore_barrier()

    def zsrc(l):
        return z_hbm.at[pl.ds(pl.multiple_of(row_base + l * K, 8), K)]

    def isrc(l):
        return b_hbm.at[pl.ds(pl.multiple_of(id_base + l * K, 8), K)]

    def wait_and_scatter(l, b):
        pltpu.make_async_copy(zsrc(l), zbufs[b], zsems[b]).wait()
        pltpu.make_async_copy(isrc(l), ibufs[b], isems[b]).wait()
        pltpu.sync_copy(zbufs[b], acc.at[ibufs[b]], add=True)

    for b in range(NBUF):
        pltpu.async_copy(zsrc(b), zbufs[b], zsems[b])
        pltpu.async_copy(isrc(b), ibufs[b], isems[b])

    def body(i, carry):
        for b in range(NBUF):
            l = i * NBUF + b
            wait_and_scatter(l, b)
            nl = l + NBUF

            @pl.when(nl < nk)
            def _next():
                pltpu.async_copy(zsrc(nl), zbufs[b], zsems[b])
                pltpu.async_copy(isrc(nl), ibufs[b], isems[b])

        return carry

    lax.fori_loop(0, nk // NBUF, body, 0)
    for l in range(nk - nk % NBUF, nk):
        wait_and_scatter(l, l % NBUF)

    if ktail:
        t0 = pl.multiple_of(row_base + nk * K, 8)
        ti0 = pl.multiple_of(id_base + nk * K, 8)
        pltpu.sync_copy(z_hbm.at[pl.ds(t0, ktail)], zb0.at[pl.ds(0, ktail)])
        pltpu.sync_copy(b_hbm.at[pl.ds(ti0, ktail)], tbuf.at[pl.ds(0, ktail)])
        pltpu.sync_copy(zb0.at[pl.ds(0, ktail)],
                        acc.at[tbuf.at[pl.ds(0, ktail)]], add=True)

    plsc.subcore_barrier()

    pltpu.sync_copy(acc.at[pl.ds(zr0, SEG_PER_TILE)],
                    out_hbm.at[c].at[pl.ds(zr0, SEG_PER_TILE)])

    @pl.when(s == NS - 1)
    def _write_tail():
        t0w = pl.multiple_of(NS * SEG_PER_TILE, 8)
        pltpu.sync_copy(acc.at[pl.ds(t0w, SEG_TAIL)],
                        out_hbm.at[c].at[pl.ds(t0w, SEG_TAIL)])


def _sc_scatter(z, bidx, zero, part):
    mesh = plsc.VectorSubcoreMesh(
        core_axis_name="c", subcore_axis_name="s", num_cores=NC, num_subcores=NS
    )
    return pl.kernel(
        functools.partial(_sc_scatter_body, part),
        out_type=jax.ShapeDtypeStruct((NC, S, D_IN), jnp.float32),
        mesh=mesh,
        scratch_types=[
            pltpu.VMEM((K, D_IN), jnp.float32),
            pltpu.VMEM((K, D_IN), jnp.float32),
            pltpu.VMEM((K, D_IN), jnp.float32),
            pltpu.VMEM((K,), jnp.int32),
            pltpu.VMEM((K,), jnp.int32),
            pltpu.VMEM((K,), jnp.int32),
            pltpu.VMEM((K,), jnp.int32),
            pltpu.VMEM_SHARED((S, D_IN), jnp.float32),
            pltpu.SemaphoreType.DMA,
            pltpu.SemaphoreType.DMA,
            pltpu.SemaphoreType.DMA,
            pltpu.SemaphoreType.DMA,
            pltpu.SemaphoreType.DMA,
            pltpu.SemaphoreType.DMA,
        ],
    )(z, bidx, zero)


def _combine1_body(p_ref, b2_ref, o_ref):
    o_ref[...] = p_ref[0] + p_ref[1] + b2_ref[...]


def _combine1(partial, b2):
    return pl.pallas_call(
        _combine1_body,
        grid=(10,),
        in_specs=[
            pl.BlockSpec((NC, S // 10, D_IN), lambda i: (0, i, 0)),
            pl.BlockSpec((1, D_IN), lambda i: (0, 0)),
        ],
        out_specs=pl.BlockSpec((S // 10, D_IN), lambda i: (i, 0)),
        out_shape=jax.ShapeDtypeStruct((S, D_IN), jnp.float32),
    )(partial, b2.reshape(1, D_IN))


def _combine2_body(h_ref, p_ref, o_ref):
    o_ref[...] = h_ref[...] + p_ref[0] + p_ref[1]


def _combine2(half, partial):
    return pl.pallas_call(
        _combine2_body,
        grid=(10,),
        in_specs=[
            pl.BlockSpec((S // 10, D_IN), lambda i: (i, 0)),
            pl.BlockSpec((NC, S // 10, D_IN), lambda i: (0, i, 0)),
        ],
        out_specs=pl.BlockSpec((S // 10, D_IN), lambda i: (i, 0)),
        out_shape=jax.ShapeDtypeStruct((S, D_IN), jnp.float32),
    )(half, partial)


def kernel(x, batch, W1, b1, W2, b2):
    bidx = batch.astype(jnp.int32)
    w1t = W1.T.astype(jnp.bfloat16)
    w2t = W2.T.astype(jnp.bfloat16)
    zero = jnp.zeros((S, D_IN), jnp.float32)
    partials = []
    for part in range(len(PARTS)):
        z = _compute_z(x, w1t, b1, w2t, part)
        partials.append(_sc_scatter(z, bidx, zero, part))
    half = _combine1(partials[0], b2)
    return _combine2(half, partials[1])


# speedup vs baseline: 1.0693x; 1.0440x over previous
import functools

import jax
import jax.numpy as jnp
from jax import lax
from jax.experimental import pallas as pl
from jax.experimental.pallas import tpu as pltpu
from jax.experimental.pallas import tpu_sc as plsc

N = 320000
D_IN = 128
D_HID = 256
S = 10000

PARTS = (160000, 160000)
RB = 16000
NC = 2
NS = 16
NW = NC * NS
K = 128
NBUF = 3
SEG_PER_TILE = 624
SEG_TAIL = S - NS * SEG_PER_TILE

PART_OFF = tuple(sum(PARTS[:i]) for i in range(len(PARTS)))
assert sum(PARTS) == N
for _p in PARTS:
    _rpw = _p // NW
    assert _p % NW == 0 and _rpw % 8 == 0 and _p % RB == 0
    assert (_rpw - (_rpw // K) * K) % 8 == 0


def _z_body(x_ref, w1t_ref, b1_ref, w2t_ref, z_ref):
    xb = x_ref[...].astype(jnp.bfloat16)
    h = jnp.dot(xb, w1t_ref[...], preferred_element_type=jnp.float32)
    phi = jnp.tanh(h + b1_ref[...])
    z_ref[...] = jnp.dot(phi.astype(jnp.bfloat16), w2t_ref[...],
                         preferred_element_type=jnp.float32)


def _compute_z(x, W1T, b1, W2T, part):
    npart = PARTS[part]
    nb0 = PART_OFF[part] // RB
    return pl.pallas_call(
        _z_body,
        grid=(npart // RB,),
        in_specs=[
            pl.BlockSpec((RB, D_IN), lambda i: (i + nb0, 0)),
            pl.BlockSpec((D_IN, D_HID), lambda i: (0, 0)),
            pl.BlockSpec((1, D_HID), lambda i: (0, 0)),
            pl.BlockSpec((D_HID, D_IN), lambda i: (0, 0)),
        ],
        out_specs=pl.BlockSpec((RB, D_IN), lambda i: (i, 0)),
        out_shape=jax.ShapeDtypeStruct((npart, D_IN), jnp.float32),
    )(x, W1T, b1.reshape(1, D_HID), W2T)


def _sc_scatter_body(part, z_hbm, b_hbm, out_hbm,
                     zb0, zb1, zb2, ib0, ib1, ib2, tbuf, acc,
                     zs0, zs1, zs2, is0, is1, is2):
    rpw = PARTS[part] // NW
    nk = rpw // K
    ktail = rpw - nk * K

    c = lax.axis_index("c")
    s = lax.axis_index("s")
    w = c * NS + s
    row_base = w * rpw
    id_base = PART_OFF[part] + w * rpw
    zbufs = (zb0, zb1, zb2)
    ibufs = (ib0, ib1, ib2)
    zsems = (zs0, zs1, zs2)
    isems = (is0, is1, is2)

    zv = jnp.zeros((16,), jnp.float32)

    def _zrow(r, carry):
        for cc in range(8):
            zb0[r, pl.ds(cc * 16, 16)] = zv
        return carry

    lax.fori_loop(0, K, _zrow, 0)
    zr0 = pl.multiple_of(s * SEG_PER_TILE, 8)
    for t in range(SEG_PER_TILE // K):
        pltpu.sync_copy(zb0, acc.at[pl.ds(zr0 + t * K, K)])
    _zrem = SEG_PER_TILE % K
    if _zrem:
        pltpu.sync_copy(zb0.at[pl.ds(0, _zrem)],
                        acc.at[pl.ds(zr0 + SEG_PER_TILE - _zrem, _zrem)])

    @pl.when(s == NS - 1)
    def _zero_tail():
        t0 = pl.multiple_of(NS * SEG_PER_TILE, 8)
        pltpu.sync_copy(zb0.at[pl.ds(0, SEG_TAIL)], acc.at[pl.ds(t0, SEG_TAIL)])

    plsc.subcore_barrier()

    def zsrc(l):
        return z_hbm.at[pl.ds(pl.multiple_of(row_base + l * K, 8), K)]

    def isrc(l):
        return b_hbm.at[pl.ds(pl.multiple_of(id_base + l * K, 8), K)]

    def wait_and_scatter(l, b):
        pltpu.make_async_copy(zsrc(l), zbufs[b], zsems[b]).wait()
        pltpu.make_async_copy(isrc(l), ibufs[b], isems[b]).wait()
        pltpu.sync_copy(zbufs[b], acc.at[ibufs[b]], add=True)

    for b in range(NBUF):
        pltpu.async_copy(zsrc(b), zbufs[b], zsems[b])
        pltpu.async_copy(isrc(b), ibufs[b], isems[b])

    def body(i, carry):
        for b in range(NBUF):
            l = i * NBUF + b
            wait_and_scatter(l, b)
            nl = l + NBUF

            @pl.when(nl < nk)
            def _next():
                pltpu.async_copy(zsrc(nl), zbufs[b], zsems[b])
                pltpu.async_copy(isrc(nl), ibufs[b], isems[b])

        return carry

    lax.fori_loop(0, nk // NBUF, body, 0)
    for l in range(nk - nk % NBUF, nk):
        wait_and_scatter(l, l % NBUF)

    if ktail:
        t0 = pl.multiple_of(row_base + nk * K, 8)
        ti0 = pl.multiple_of(id_base + nk * K, 8)
        pltpu.sync_copy(z_hbm.at[pl.ds(t0, ktail)], zb0.at[pl.ds(0, ktail)])
        pltpu.sync_copy(b_hbm.at[pl.ds(ti0, ktail)], tbuf.at[pl.ds(0, ktail)])
        pltpu.sync_copy(zb0.at[pl.ds(0, ktail)],
                        acc.at[tbuf.at[pl.ds(0, ktail)]], add=True)

    plsc.subcore_barrier()

    pltpu.sync_copy(acc.at[pl.ds(zr0, SEG_PER_TILE)],
                    out_hbm.at[c].at[pl.ds(zr0, SEG_PER_TILE)])

    @pl.when(s == NS - 1)
    def _write_tail():
        t0w = pl.multiple_of(NS * SEG_PER_TILE, 8)
        pltpu.sync_copy(acc.at[pl.ds(t0w, SEG_TAIL)],
                        out_hbm.at[c].at[pl.ds(t0w, SEG_TAIL)])


def _sc_scatter(z, bidx, part):
    mesh = plsc.VectorSubcoreMesh(
        core_axis_name="c", subcore_axis_name="s", num_cores=NC, num_subcores=NS
    )
    return pl.kernel(
        functools.partial(_sc_scatter_body, part),
        out_type=jax.ShapeDtypeStruct((NC, S, D_IN), jnp.float32),
        mesh=mesh,
        scratch_types=[
            pltpu.VMEM((K, D_IN), jnp.float32),
            pltpu.VMEM((K, D_IN), jnp.float32),
            pltpu.VMEM((K, D_IN), jnp.float32),
            pltpu.VMEM((K,), jnp.int32),
            pltpu.VMEM((K,), jnp.int32),
            pltpu.VMEM((K,), jnp.int32),
            pltpu.VMEM((K,), jnp.int32),
            pltpu.VMEM_SHARED((S, D_IN), jnp.float32),
            pltpu.SemaphoreType.DMA,
            pltpu.SemaphoreType.DMA,
            pltpu.SemaphoreType.DMA,
            pltpu.SemaphoreType.DMA,
            pltpu.SemaphoreType.DMA,
            pltpu.SemaphoreType.DMA,
        ],
    )(z, bidx)


def _combine1_body(p_ref, b2_ref, o_ref):
    o_ref[...] = p_ref[0] + p_ref[1] + b2_ref[...]


def _combine1(partial, b2):
    return pl.pallas_call(
        _combine1_body,
        grid=(10,),
        in_specs=[
            pl.BlockSpec((NC, S // 10, D_IN), lambda i: (0, i, 0)),
            pl.BlockSpec((1, D_IN), lambda i: (0, 0)),
        ],
        out_specs=pl.BlockSpec((S // 10, D_IN), lambda i: (i, 0)),
        out_shape=jax.ShapeDtypeStruct((S, D_IN), jnp.float32),
    )(partial, b2.reshape(1, D_IN))


def _combine2_body(h_ref, p_ref, o_ref):
    o_ref[...] = h_ref[...] + p_ref[0] + p_ref[1]


def _combine2(half, partial):
    return pl.pallas_call(
        _combine2_body,
        grid=(10,),
        in_specs=[
            pl.BlockSpec((S // 10, D_IN), lambda i: (i, 0)),
            pl.BlockSpec((NC, S // 10, D_IN), lambda i: (0, i, 0)),
        ],
        out_specs=pl.BlockSpec((S // 10, D_IN), lambda i: (i, 0)),
        out_shape=jax.ShapeDtypeStruct((S, D_IN), jnp.float32),
    )(half, partial)


def kernel(x, batch, W1, b1, W2, b2):
    bidx = batch.astype(jnp.int32)
    w1t = W1.T.astype(jnp.bfloat16)
    w2t = W2.T.astype(jnp.bfloat16)
    partials = []
    for part in range(len(PARTS)):
        z = _compute_z(x, w1t, b1, w2t, part)
        partials.append(_sc_scatter(z, bidx, part))
    half = _combine1(partials[0], b2)
    return _combine2(half, partials[1])


# speedup vs baseline: 1.0710x; 1.0016x over previous
import functools

import jax
import jax.numpy as jnp
from jax import lax
from jax.experimental import pallas as pl
from jax.experimental.pallas import tpu as pltpu
from jax.experimental.pallas import tpu_sc as plsc

N = 320000
D_IN = 128
D_HID = 256
S = 10000

PARTS = (160000, 160000)
RB = 20000
NC = 2
NS = 16
NW = NC * NS
K = 128
NBUF = 3
SEG_PER_TILE = 624
SEG_TAIL = S - NS * SEG_PER_TILE

PART_OFF = tuple(sum(PARTS[:i]) for i in range(len(PARTS)))
assert sum(PARTS) == N
for _p in PARTS:
    _rpw = _p // NW
    assert _p % NW == 0 and _rpw % 8 == 0 and _p % RB == 0
    assert (_rpw - (_rpw // K) * K) % 8 == 0


def _z_body(x_ref, w1t_ref, b1_ref, w2t_ref, z_ref):
    xb = x_ref[...].astype(jnp.bfloat16)
    h = jnp.dot(xb, w1t_ref[...], preferred_element_type=jnp.float32)
    phi = jnp.tanh(h + b1_ref[...])
    z_ref[...] = jnp.dot(phi.astype(jnp.bfloat16), w2t_ref[...],
                         preferred_element_type=jnp.float32)


def _compute_z(x, W1T, b1, W2T, part):
    npart = PARTS[part]
    nb0 = PART_OFF[part] // RB
    return pl.pallas_call(
        _z_body,
        grid=(npart // RB,),
        in_specs=[
            pl.BlockSpec((RB, D_IN), lambda i: (i + nb0, 0)),
            pl.BlockSpec((D_IN, D_HID), lambda i: (0, 0)),
            pl.BlockSpec((1, D_HID), lambda i: (0, 0)),
            pl.BlockSpec((D_HID, D_IN), lambda i: (0, 0)),
        ],
        out_specs=pl.BlockSpec((RB, D_IN), lambda i: (i, 0)),
        out_shape=jax.ShapeDtypeStruct((npart, D_IN), jnp.float32),
    )(x, W1T, b1.reshape(1, D_HID), W2T)


def _sc_scatter_body(part, z_hbm, b_hbm, out_hbm,
                     zb0, zb1, zb2, ib0, ib1, ib2, tbuf, acc,
                     zs0, zs1, zs2, is0, is1, is2):
    rpw = PARTS[part] // NW
    nk = rpw // K
    ktail = rpw - nk * K

    c = lax.axis_index("c")
    s = lax.axis_index("s")
    w = c * NS + s
    row_base = w * rpw
    id_base = PART_OFF[part] + w * rpw
    zbufs = (zb0, zb1, zb2)
    ibufs = (ib0, ib1, ib2)
    zsems = (zs0, zs1, zs2)
    isems = (is0, is1, is2)

    zv = jnp.zeros((16,), jnp.float32)

    def _zrow(r, carry):
        for cc in range(8):
            zb0[r, pl.ds(cc * 16, 16)] = zv
        return carry

    lax.fori_loop(0, K, _zrow, 0)
    zr0 = pl.multiple_of(s * SEG_PER_TILE, 8)
    for t in range(SEG_PER_TILE // K):
        pltpu.sync_copy(zb0, acc.at[pl.ds(zr0 + t * K, K)])
    _zrem = SEG_PER_TILE % K
    if _zrem:
        pltpu.sync_copy(zb0.at[pl.ds(0, _zrem)],
                        acc.at[pl.ds(zr0 + SEG_PER_TILE - _zrem, _zrem)])

    @pl.when(s == NS - 1)
    def _zero_tail():
        t0 = pl.multiple_of(NS * SEG_PER_TILE, 8)
        pltpu.sync_copy(zb0.at[pl.ds(0, SEG_TAIL)], acc.at[pl.ds(t0, SEG_TAIL)])

    plsc.subcore_barrier()

    def zsrc(l):
        return z_hbm.at[pl.ds(pl.multiple_of(row_base + l * K, 8), K)]

    def isrc(l):
        return b_hbm.at[pl.ds(pl.multiple_of(id_base + l * K, 8), K)]

    def wait_and_scatter(l, b):
        pltpu.make_async_copy(zsrc(l), zbufs[b], zsems[b]).wait()
        pltpu.make_async_copy(isrc(l), ibufs[b], isems[b]).wait()
        pltpu.sync_copy(zbufs[b], acc.at[ibufs[b]], add=True)

    for b in range(NBUF):
        pltpu.async_copy(zsrc(b), zbufs[b], zsems[b])
        pltpu.async_copy(isrc(b), ibufs[b], isems[b])

    def body(i, carry):
        for b in range(NBUF):
            l = i * NBUF + b
            wait_and_scatter(l, b)
            nl = l + NBUF

            @pl.when(nl < nk)
            def _next():
                pltpu.async_copy(zsrc(nl), zbufs[b], zsems[b])
                pltpu.async_copy(isrc(nl), ibufs[b], isems[b])

        return carry

    lax.fori_loop(0, nk // NBUF, body, 0)
    for l in range(nk - nk % NBUF, nk):
        wait_and_scatter(l, l % NBUF)

    if ktail:
        t0 = pl.multiple_of(row_base + nk * K, 8)
        ti0 = pl.multiple_of(id_base + nk * K, 8)
        pltpu.sync_copy(z_hbm.at[pl.ds(t0, ktail)], zb0.at[pl.ds(0, ktail)])
        pltpu.sync_copy(b_hbm.at[pl.ds(ti0, ktail)], tbuf.at[pl.ds(0, ktail)])
        pltpu.sync_copy(zb0.at[pl.ds(0, ktail)],
                        acc.at[tbuf.at[pl.ds(0, ktail)]], add=True)

    plsc.subcore_barrier()

    pltpu.sync_copy(acc.at[pl.ds(zr0, SEG_PER_TILE)],
                    out_hbm.at[c].at[pl.ds(zr0, SEG_PER_TILE)])

    @pl.when(s == NS - 1)
    def _write_tail():
        t0w = pl.multiple_of(NS * SEG_PER_TILE, 8)
        pltpu.sync_copy(acc.at[pl.ds(t0w, SEG_TAIL)],
                        out_hbm.at[c].at[pl.ds(t0w, SEG_TAIL)])


def _sc_scatter(z, bidx, part):
    mesh = plsc.VectorSubcoreMesh(
        core_axis_name="c", subcore_axis_name="s", num_cores=NC, num_subcores=NS
    )
    return pl.kernel(
        functools.partial(_sc_scatter_body, part),
        out_type=jax.ShapeDtypeStruct((NC, S, D_IN), jnp.float32),
        mesh=mesh,
        scratch_types=[
            pltpu.VMEM((K, D_IN), jnp.float32),
            pltpu.VMEM((K, D_IN), jnp.float32),
            pltpu.VMEM((K, D_IN), jnp.float32),
            pltpu.VMEM((K,), jnp.int32),
            pltpu.VMEM((K,), jnp.int32),
            pltpu.VMEM((K,), jnp.int32),
            pltpu.VMEM((K,), jnp.int32),
            pltpu.VMEM_SHARED((S, D_IN), jnp.float32),
            pltpu.SemaphoreType.DMA,
            pltpu.SemaphoreType.DMA,
            pltpu.SemaphoreType.DMA,
            pltpu.SemaphoreType.DMA,
            pltpu.SemaphoreType.DMA,
            pltpu.SemaphoreType.DMA,
        ],
    )(z, bidx)


def _combine1_body(p_ref, b2_ref, o_ref):
    o_ref[...] = p_ref[0] + p_ref[1] + b2_ref[...]


def _combine1(partial, b2):
    return pl.pallas_call(
        _combine1_body,
        grid=(10,),
        in_specs=[
            pl.BlockSpec((NC, S // 10, D_IN), lambda i: (0, i, 0)),
            pl.BlockSpec((1, D_IN), lambda i: (0, 0)),
        ],
        out_specs=pl.BlockSpec((S // 10, D_IN), lambda i: (i, 0)),
        out_shape=jax.ShapeDtypeStruct((S, D_IN), jnp.float32),
    )(partial, b2.reshape(1, D_IN))


def _combine2_body(h_ref, p_ref, o_ref):
    o_ref[...] = h_ref[...] + p_ref[0] + p_ref[1]


def _combine2(half, partial):
    return pl.pallas_call(
        _combine2_body,
        grid=(10,),
        in_specs=[
            pl.BlockSpec((S // 10, D_IN), lambda i: (i, 0)),
            pl.BlockSpec((NC, S // 10, D_IN), lambda i: (0, i, 0)),
        ],
        out_specs=pl.BlockSpec((S // 10, D_IN), lambda i: (i, 0)),
        out_shape=jax.ShapeDtypeStruct((S, D_IN), jnp.float32),
    )(half, partial)


def kernel(x, batch, W1, b1, W2, b2):
    bidx = batch.astype(jnp.int32)
    w1t = W1.T.astype(jnp.bfloat16)
    w2t = W2.T.astype(jnp.bfloat16)
    partials = []
    for part in range(len(PARTS)):
        z = _compute_z(x, w1t, b1, w2t, part)
        partials.append(_sc_scatter(z, bidx, part))
    half = _combine1(partials[0], b2)
    return _combine2(half, partials[1])
